# baseline (speedup 1.0000x reference)
import functools

import jax
import jax.numpy as jnp
from jax.experimental import pallas as pl
from jax.experimental.pallas import tpu as pltpu


def _support_body(x_ref, w_ref, out_ref):
    out_ref[:] = jnp.dot(
        x_ref[:], w_ref[:], preferred_element_type=jnp.float32
    ).astype(jnp.bfloat16)


def _agg_body(adj_ref, supp_ref, out_ref):
    acc = jnp.dot(
        adj_ref[:].astype(jnp.bfloat16),
        supp_ref[:],
        preferred_element_type=jnp.float32,
    )
    out_ref[:] = jnp.maximum(acc, 0.0)


@functools.partial(jax.jit, static_argnames=("block_m",))
def _gcn(x, adj, W, block_m):
    n, d_in = x.shape
    d_hid = W.shape[1]
    supp = pl.pallas_call(
        _support_body,
        in_specs=[
            pl.BlockSpec((n, d_in), lambda: (0, 0)),
            pl.BlockSpec((d_in, d_hid), lambda: (0, 0)),
        ],
        out_specs=pl.BlockSpec((n, d_hid), lambda: (0, 0)),
        out_shape=jax.ShapeDtypeStruct((n, d_hid), jnp.bfloat16),
    )(x, W)
    return pl.pallas_call(
        _agg_body,
        grid=(n // block_m,),
        in_specs=[
            pl.BlockSpec((block_m, n), lambda i: (i, 0)),
            pl.BlockSpec((n, d_hid), lambda i: (0, 0)),
        ],
        out_specs=pl.BlockSpec((block_m, d_hid), lambda i: (i, 0)),
        out_shape=jax.ShapeDtypeStruct((n, d_hid), jnp.float32),
        compiler_params=pltpu.CompilerParams(
            dimension_semantics=("parallel",),
        ),
    )(adj, supp)


def kernel(x, adj, W):
    return _gcn(x, adj, W, 400)


# speedup vs baseline: 1.0138x; 1.0138x over previous
import functools

import jax
import jax.numpy as jnp
from jax.experimental import pallas as pl
from jax.experimental.pallas import tpu as pltpu


def _gcn_body(x_ref, adja_ref, adjb_ref, w_ref, out_ref, supp_ref):
    @pl.when(pl.program_id(0) == 0)
    def _():
        supp_ref[:] = jnp.dot(
            x_ref[:], w_ref[:], preferred_element_type=jnp.float32
        ).astype(jnp.bfloat16)

    supp = supp_ref[:]
    out_ref[0] = jnp.maximum(
        jnp.dot(adja_ref[0].astype(jnp.bfloat16), supp,
                preferred_element_type=jnp.float32),
        0.0,
    )
    out_ref[1] = jnp.maximum(
        jnp.dot(adjb_ref[0].astype(jnp.bfloat16), supp,
                preferred_element_type=jnp.float32),
        0.0,
    )


@functools.partial(jax.jit, static_argnames=("block_m",))
def _gcn(x, adj, W, block_m):
    n, d_in = x.shape
    d_hid = W.shape[1]
    half = n // 2
    adj3 = adj.reshape(2, half, n)
    grid = (half // block_m,)
    out = pl.pallas_call(
        _gcn_body,
        grid=grid,
        in_specs=[
            pl.BlockSpec((n, d_in), lambda i: (0, 0, )),
            pl.BlockSpec((1, block_m, n), lambda i: (0, i, 0)),
            pl.BlockSpec((1, block_m, n), lambda i: (1, i, 0)),
            pl.BlockSpec((d_in, d_hid), lambda i: (0, 0)),
        ],
        out_specs=pl.BlockSpec((2, block_m, d_hid), lambda i: (0, i, 0)),
        out_shape=jax.ShapeDtypeStruct((2, half, d_hid), jnp.float32),
        scratch_shapes=[pltpu.VMEM((n, d_hid), jnp.bfloat16)],
        compiler_params=pltpu.CompilerParams(
            dimension_semantics=("arbitrary",),
        ),
    )(x, adj3, adj3, W)
    return out.reshape(n, d_hid)


def kernel(x, adj, W):
    return _gcn(x, adj, W, 200)


# speedup vs baseline: 1.0274x; 1.0134x over previous
import functools

import jax
import jax.numpy as jnp
from jax.experimental import pallas as pl
from jax.experimental.pallas import tpu as pltpu


def _gcn_body(x_ref, adj_ref, w_ref, out_ref, supp_ref):
    @pl.when(pl.program_id(0) == 0)
    def _():
        supp_ref[:] = jnp.dot(
            x_ref[:], w_ref[:], preferred_element_type=jnp.float32
        ).astype(jnp.bfloat16)

    acc = jnp.dot(
        adj_ref[:].astype(jnp.bfloat16),
        supp_ref[:],
        preferred_element_type=jnp.float32,
    )
    out_ref[:] = jnp.maximum(acc, 0.0)


@functools.partial(jax.jit, static_argnames=("block_m",))
def _gcn(x, adj, W, block_m):
    n, d_in = x.shape
    d_hid = W.shape[1]
    grid = (n // block_m,)
    return pl.pallas_call(
        _gcn_body,
        grid=grid,
        in_specs=[
            pl.BlockSpec((n, d_in), lambda i: (0, 0)),
            pl.BlockSpec((block_m, n), lambda i: (i, 0)),
            pl.BlockSpec((d_in, d_hid), lambda i: (0, 0)),
        ],
        out_specs=pl.BlockSpec((block_m, d_hid), lambda i: (i, 0)),
        out_shape=jax.ShapeDtypeStruct((n, d_hid), jnp.float32),
        scratch_shapes=[pltpu.VMEM((n, d_hid), jnp.bfloat16)],
        compiler_params=pltpu.CompilerParams(
            dimension_semantics=("arbitrary",),
        ),
    )(x, adj, W)


def kernel(x, adj, W):
    return _gcn(x, adj, W, 400)
